# baseline (speedup 1.0000x reference)
import jax
import jax.numpy as jnp
from jax.experimental import pallas as pl
from jax.experimental.pallas import tpu as pltpu

_TEMPERATURE = 2.0
_BLOCK_T = 1024
_NBUF = 3
_STRIPES = 4
_ROWS = _BLOCK_T // _STRIPES


def _router_body(x_hbm, wt_ref, b_ref, out_hbm, xbuf, obuf, sems, osems):
    tokens = x_hbm.shape[0]
    nchunks = tokens // _BLOCK_T

    def stripe_copy(i, slot, s):
        return pltpu.make_async_copy(
            x_hbm.at[pl.ds(i * _BLOCK_T + s * _ROWS, _ROWS), :],
            xbuf.at[slot, pl.ds(s * _ROWS, _ROWS), :],
            sems.at[slot, s],
        )

    def start(i, slot):
        for s in range(_STRIPES):
            stripe_copy(i, slot, s).start()

    def wait(i, slot):
        for s in range(_STRIPES):
            stripe_copy(i, slot, s).wait()

    def out_copy(i, oslot):
        return pltpu.make_async_copy(
            obuf.at[oslot],
            out_hbm.at[pl.ds(i * _BLOCK_T, _BLOCK_T), :],
            osems.at[oslot],
        )

    for k in range(_NBUF):
        start(k, k)

    def step(i, carry):
        slot = jax.lax.rem(i, _NBUF)
        wait(i, slot)
        logits = jnp.dot(xbuf[slot], wt_ref[...], preferred_element_type=jnp.float32)
        logits = (logits + b_ref[...]) * (1.0 / _TEMPERATURE)
        m = jnp.max(logits, axis=-1, keepdims=True)
        e = jnp.exp(logits - m)
        probs = e / jnp.sum(e, axis=-1, keepdims=True)

        oslot = jax.lax.rem(i, 2)

        @pl.when(i >= 2)
        def _():
            out_copy(i - 2, oslot).wait()

        obuf[oslot] = probs
        out_copy(i, oslot).start()

        @pl.when(i + _NBUF < nchunks)
        def _():
            start(i + _NBUF, slot)

        return carry

    jax.lax.fori_loop(0, nchunks, step, 0)
    out_copy(nchunks - 2, jax.lax.rem(nchunks - 2, 2)).wait()
    out_copy(nchunks - 1, jax.lax.rem(nchunks - 1, 2)).wait()


def kernel(x, W, b):
    tokens, d_model = x.shape
    num_experts = W.shape[0]
    wt = W.T
    b2 = b.reshape(1, num_experts)
    return pl.pallas_call(
        _router_body,
        in_specs=[
            pl.BlockSpec(memory_space=pl.ANY),
            pl.BlockSpec((d_model, num_experts), lambda: (0, 0)),
            pl.BlockSpec((1, num_experts), lambda: (0, 0)),
        ],
        out_specs=pl.BlockSpec(memory_space=pl.ANY),
        out_shape=jax.ShapeDtypeStruct((tokens, num_experts), jnp.float32),
        scratch_shapes=[
            pltpu.VMEM((_NBUF, _BLOCK_T, d_model), jnp.float32),
            pltpu.VMEM((2, _BLOCK_T, num_experts), jnp.float32),
            pltpu.SemaphoreType.DMA((_NBUF, _STRIPES)),
            pltpu.SemaphoreType.DMA((2,)),
        ],
    )(x, wt, b2)


# speedup vs baseline: 1.0237x; 1.0237x over previous
import jax
import jax.numpy as jnp
from jax.experimental import pallas as pl
from jax.experimental.pallas import tpu as pltpu

_TEMPERATURE = 2.0
_BLOCK_T = 1024


def _router_block(x_ref, wt_ref, b_ref, out_ref):
    xb = x_ref[...].astype(jnp.bfloat16)
    logits = jnp.dot(xb, wt_ref[...], preferred_element_type=jnp.float32)
    logits = (logits + b_ref[...]) * (1.0 / _TEMPERATURE)
    m = jnp.max(logits, axis=-1, keepdims=True)
    e = jnp.exp(logits - m)
    out_ref[...] = e / jnp.sum(e, axis=-1, keepdims=True)


def kernel(x, W, b):
    tokens, d_model = x.shape
    num_experts = W.shape[0]
    wt = W.T.astype(jnp.bfloat16)
    b2 = b.reshape(1, num_experts)
    bt = _BLOCK_T
    return pl.pallas_call(
        _router_block,
        grid=(tokens // bt,),
        in_specs=[
            pl.BlockSpec((bt, d_model), lambda i: (i, 0)),
            pl.BlockSpec((d_model, num_experts), lambda i: (0, 0)),
            pl.BlockSpec((1, num_experts), lambda i: (0, 0)),
        ],
        out_specs=pl.BlockSpec((bt, num_experts), lambda i: (i, 0)),
        out_shape=jax.ShapeDtypeStruct((tokens, num_experts), jnp.float32),
        compiler_params=pltpu.CompilerParams(
            dimension_semantics=("arbitrary",),
        ),
    )(x, wt, b2)
